# baseline (speedup 1.0000x reference)
import functools
import math

import jax
import jax.numpy as jnp
from jax import lax
from jax.experimental import pallas as pl
from jax.experimental.pallas import tpu as pltpu
from jax.experimental.pallas import tpu_sc as plsc

_NC, _NS = 2, 16
_NW = _NC * _NS


def _leaky(x):
    return jnp.where(x >= 0, x, 0.1 * x)


def _emb_body(fb_ref, we_ref, be_ref, bg_ref, msg_ref, bgt_ref):
    x = jnp.dot(fb_ref[...], we_ref[...], preferred_element_type=jnp.float32)
    msg_ref[...] = _leaky(x + be_ref[...])
    bgt_ref[...] = bg_ref[...].T


def _make_gather(total, d, chunk, dtype):
    per_w = total // _NW
    n_it = per_w // chunk
    assert per_w * _NW == total and n_it * chunk == per_w and n_it >= 2
    mesh = plsc.VectorSubcoreMesh(core_axis_name="c", subcore_axis_name="s",
                                  num_cores=_NC, num_subcores=_NS)

    @functools.partial(
        pl.kernel, mesh=mesh,
        out_type=jax.ShapeDtypeStruct((total, d), dtype),
        scratch_types=[pltpu.VMEM((chunk,), jnp.int32),
                       pltpu.VMEM((chunk,), jnp.int32),
                       pltpu.VMEM((chunk, d), dtype),
                       pltpu.VMEM((chunk, d), dtype),
                       pltpu.SemaphoreType.DMA,
                       pltpu.SemaphoreType.DMA,
                       pltpu.SemaphoreType.DMA,
                       pltpu.SemaphoreType.DMA],
    )
    def gather(tbl_hbm, idx_hbm, out_hbm, i0, i1, r0, r1, g0, g1, w0, w1):
        wid = lax.axis_index("s") * _NC + lax.axis_index("c")
        base = wid * per_w
        idxs, rows, gs, ws = (i0, i1), (r0, r1), (g0, g1), (w0, w1)
        for b in range(2):
            pltpu.sync_copy(idx_hbm.at[pl.ds(base + b * chunk, chunk)], idxs[b])
            pltpu.async_copy(tbl_hbm.at[idxs[b]], rows[b], gs[b])

        def body(i, carry):
            for b in range(2):
                cur = 2 * i + b

                @pl.when(cur < n_it)
                def _():
                    off = base + cur * chunk
                    pltpu.make_async_copy(tbl_hbm.at[idxs[b]], rows[b],
                                          gs[b]).wait()
                    pltpu.async_copy(rows[b], out_hbm.at[pl.ds(off, chunk)],
                                     ws[b])

                    @pl.when(cur + 2 < n_it)
                    def _():
                        nxt_off = base + (cur + 2) * chunk
                        pltpu.sync_copy(idx_hbm.at[pl.ds(nxt_off, chunk)],
                                        idxs[b])
                        pltpu.make_async_copy(
                            rows[b], out_hbm.at[pl.ds(off, chunk)],
                            ws[b]).wait()
                        pltpu.async_copy(tbl_hbm.at[idxs[b]], rows[b], gs[b])

                    @pl.when(cur + 2 >= n_it)
                    def _():
                        pltpu.make_async_copy(
                            rows[b], out_hbm.at[pl.ds(off, chunk)],
                            ws[b]).wait()
            return carry

        lax.fori_loop(0, (n_it + 1) // 2, body, 0)

    return gather


def _att_body(*refs, nei, h, kd):
    msg_ref = refs[0]
    nrefs = refs[1:1 + nei]
    bg_ref = refs[1 + nei]
    wq_ref, wk_ref, wv_ref, wrep_ref = refs[2 + nei:6 + nei]
    out_ref = refs[6 + nei]
    m = msg_ref[...]
    fb_in = jnp.where(m >= 0, m, 10.0 * m)
    q = jnp.dot(m.astype(jnp.bfloat16), wq_ref[...],
                preferred_element_type=jnp.float32)
    cols = lax.broadcasted_iota(jnp.int32, (h * kd, h), 0) // kd
    hh = lax.broadcasted_iota(jnp.int32, (h * kd, h), 1)
    s_mat = (cols == hh).astype(jnp.float32)
    s_t = (lax.broadcasted_iota(jnp.int32, (h, h * kd), 1) // kd
           == lax.broadcasted_iota(jnp.int32, (h, h * kd), 0)).astype(jnp.float32)
    bg = bg_ref[...]
    inv_sqrt = 1.0 / math.sqrt(kd)
    es, vs = [], []
    ssum = None
    for j in range(nei):
        nb = nrefs[j][...].astype(jnp.bfloat16)
        kj = jnp.dot(nb, wk_ref[...], preferred_element_type=jnp.float32)
        vj = jnp.dot(nb, wv_ref[...], preferred_element_type=jnp.float32)
        lj = jnp.dot(q * kj, s_mat, preferred_element_type=jnp.float32)
        mj = (bg[:, j:j + 1] != 0).astype(jnp.float32)
        ej = jnp.exp(lj * inv_sqrt) * mj
        ssum = ej if ssum is None else ssum + ej
        es.append(ej)
        vs.append(vj)
    inv = 1.0 / jnp.maximum(ssum, 1e-30)
    wsum = None
    for j in range(nei):
        aj = jnp.dot(es[j] * inv, s_t, preferred_element_type=jnp.float32)
        term = vs[j] * aj
        wsum = term if wsum is None else wsum + term
    nm = jnp.dot(wsum, wrep_ref[...], preferred_element_type=jnp.float32)
    out_ref[...] = _leaky(fb_in + nm)


def _make_gather_sum(n_pad, d, nei, chunk_atoms):
    per_w = n_pad // _NW
    n_it = per_w // chunk_atoms
    assert per_w * _NW == n_pad and n_it * chunk_atoms == per_w
    rows_chunk = chunk_atoms * nei
    mesh = plsc.VectorSubcoreMesh(core_axis_name="c", subcore_axis_name="s",
                                  num_cores=_NC, num_subcores=_NS)

    @functools.partial(
        pl.kernel, mesh=mesh,
        out_type=jax.ShapeDtypeStruct((n_pad, d), jnp.float32),
        scratch_types=[pltpu.VMEM((rows_chunk,), jnp.int32),
                       pltpu.VMEM((rows_chunk, d), jnp.float32),
                       pltpu.VMEM((chunk_atoms, d), jnp.float32),
                       pltpu.SemaphoreType.DMA],
    )
    def gsum(tbl_hbm, idx_hbm, out_hbm, idx_v, rows_v, acc_v, sem):
        wid = lax.axis_index("s") * _NC + lax.axis_index("c")

        def outer(it, carry):
            a_off = wid * per_w + it * chunk_atoms
            pltpu.sync_copy(idx_hbm.at[pl.ds(a_off * nei, rows_chunk)], idx_v)
            pltpu.async_copy(tbl_hbm.at[idx_v], rows_v, sem).wait()

            def atom_body(a, c2):
                r = a * nei
                for f in range(d // 16):
                    sl = pl.ds(f * 16, 16)
                    acc = rows_v[r, sl]
                    for j in range(1, nei):
                        acc = acc + rows_v[r + j, sl]
                    acc_v[a, sl] = acc
                return c2

            lax.fori_loop(0, chunk_atoms, atom_body, 0)
            pltpu.sync_copy(acc_v, out_hbm.at[pl.ds(a_off, chunk_atoms)])
            return carry

        lax.fori_loop(0, n_it, outer, 0)

    return gsum


def _out_body(fat_ref, na_ref, wa_ref, wn_ref, bl_ref, out_ref):
    x = jnp.dot(fat_ref[...], wa_ref[...], preferred_element_type=jnp.float32)
    x = x + jnp.dot(na_ref[...], wn_ref[...], preferred_element_type=jnp.float32)
    out_ref[...] = _leaky(x + bl_ref[...])


def kernel(fatoms, fbonds, agraph, bgraph, lig_scope,
           W_emb, b_emb, W_Q, W_K, W_V, W_out, W_last, b_last):
    del lig_scope
    e, fb_dim = fbonds.shape
    n, fa_dim = fatoms.shape
    nei = bgraph.shape[1]
    d = W_emb.shape[1]
    kd = W_out.shape[0]
    h = W_Q.shape[1] // kd
    bgraph = bgraph.astype(jnp.int32)
    agraph = agraph.astype(jnp.int32)

    b1 = 3200
    message, bgt = pl.pallas_call(
        _emb_body,
        grid=(e // b1,),
        in_specs=[pl.BlockSpec((b1, fb_dim), lambda i: (i, 0)),
                  pl.BlockSpec((fb_dim, d), lambda i: (0, 0)),
                  pl.BlockSpec((1, d), lambda i: (0, 0)),
                  pl.BlockSpec((b1, nei), lambda i: (i, 0))],
        out_specs=[pl.BlockSpec((b1, d), lambda i: (i, 0)),
                   pl.BlockSpec((nei, b1), lambda i: (0, i))],
        out_shape=[jax.ShapeDtypeStruct((e, d), jnp.float32),
                   jax.ShapeDtypeStruct((nei, e), jnp.int32)],
        compiler_params=pltpu.CompilerParams(dimension_semantics=("parallel",)),
    )(fbonds, W_emb, b_emb.reshape(1, d), bgraph)

    b2 = 1600
    wrep = jnp.tile(W_out, (h, 1)) / h
    att = functools.partial(_att_body, nei=nei, h=h, kd=kd)
    wqb = W_Q.astype(jnp.bfloat16)
    wkb = W_K.astype(jnp.bfloat16)
    wvb = W_V.astype(jnp.bfloat16)
    he = e // 2
    gather_half = _make_gather(he * nei, d, 200, jnp.float32)
    nei_halves = [gather_half(message,
                              lax.slice_in_dim(bgt, k * he, (k + 1) * he,
                                               axis=1).reshape(-1))
                  for k in range(2)]

    def att_half(nei_rows, msg_h, bg_h):
        nblk = he // b2
        nei_specs = [pl.BlockSpec((b2, d),
                                  functools.partial(
                                      lambda j, i: (j * nblk + i, 0), j))
                     for j in range(nei)]
        return pl.pallas_call(
            att,
            grid=(nblk,),
            in_specs=[pl.BlockSpec((b2, d), lambda i: (i, 0))]
                     + nei_specs
                     + [pl.BlockSpec((b2, nei), lambda i: (i, 0)),
                        pl.BlockSpec((d, h * kd), lambda i: (0, 0)),
                        pl.BlockSpec((d, h * kd), lambda i: (0, 0)),
                        pl.BlockSpec((d, h * kd), lambda i: (0, 0)),
                        pl.BlockSpec((h * kd, d), lambda i: (0, 0))],
            out_specs=pl.BlockSpec((b2, d), lambda i: (i, 0)),
            out_shape=jax.ShapeDtypeStruct((he, d), jnp.float32),
            compiler_params=pltpu.CompilerParams(
                dimension_semantics=("parallel",)),
        )(msg_h, *([nei_rows] * nei), bg_h, wqb, wkb, wvb, wrep)

    message2 = jnp.concatenate(
        [att_half(nei_halves[k],
                  lax.slice_in_dim(message, k * he, (k + 1) * he),
                  lax.slice_in_dim(bgraph, k * he, (k + 1) * he))
         for k in range(2)], axis=0)

    n_pad = 10240
    ag_pad = jnp.zeros((n_pad, nei), jnp.int32).at[:n].set(agraph)
    nei_atom = _make_gather_sum(n_pad, d, nei, 80)(message2, ag_pad.reshape(-1))

    b3 = 1000
    out = pl.pallas_call(
        _out_body,
        grid=(n // b3,),
        in_specs=[pl.BlockSpec((b3, fa_dim), lambda i: (i, 0)),
                  pl.BlockSpec((b3, d), lambda i: (i, 0)),
                  pl.BlockSpec((fa_dim, d), lambda i: (0, 0)),
                  pl.BlockSpec((d, d), lambda i: (0, 0)),
                  pl.BlockSpec((1, d), lambda i: (0, 0))],
        out_specs=pl.BlockSpec((b3, d), lambda i: (i, 0)),
        out_shape=jax.ShapeDtypeStruct((n, d), jnp.float32),
        compiler_params=pltpu.CompilerParams(dimension_semantics=("parallel",)),
    )(fatoms, nei_atom[:n], W_last[:fa_dim], W_last[fa_dim:], b_last.reshape(1, d))
    return out


# speedup vs baseline: 1.1151x; 1.1151x over previous
import functools
import math

import jax
import jax.numpy as jnp
from jax import lax
from jax.experimental import pallas as pl
from jax.experimental.pallas import tpu as pltpu
from jax.experimental.pallas import tpu_sc as plsc

_NC, _NS = 2, 16
_NW = _NC * _NS


def _leaky(x):
    return jnp.where(x >= 0, x, 0.1 * x)


def _emb_body(fb_ref, we_ref, be_ref, bg_ref, msg_ref, bgt_ref):
    x = jnp.dot(fb_ref[...], we_ref[...], preferred_element_type=jnp.float32)
    msg_ref[...] = _leaky(x + be_ref[...])
    bgt_ref[...] = bg_ref[...].T


def _make_gather(total, d, chunk, dtype, seg_out=None, seg_idx=None, base=0):
    if seg_out is None:
        seg_out, seg_idx = total, total
    per_w = total // _NW
    n_it = per_w // chunk
    assert per_w * _NW == total and n_it * chunk == per_w and n_it >= 2
    assert seg_out % chunk == 0 and base % 8 == 0 and seg_idx % 8 == 0
    mesh = plsc.VectorSubcoreMesh(core_axis_name="c", subcore_axis_name="s",
                                  num_cores=_NC, num_subcores=_NS)

    @functools.partial(
        pl.kernel, mesh=mesh,
        out_type=jax.ShapeDtypeStruct((total, d), dtype),
        scratch_types=[pltpu.VMEM((chunk,), jnp.int32),
                       pltpu.VMEM((chunk,), jnp.int32),
                       pltpu.VMEM((chunk, d), dtype),
                       pltpu.VMEM((chunk, d), dtype),
                       pltpu.SemaphoreType.DMA,
                       pltpu.SemaphoreType.DMA,
                       pltpu.SemaphoreType.DMA,
                       pltpu.SemaphoreType.DMA],
    )
    def gather(tbl_hbm, idx_hbm, out_hbm, i0, i1, r0, r1, g0, g1, w0, w1):
        wid = lax.axis_index("s") * _NC + lax.axis_index("c")
        wbase = wid * per_w
        idxs, rows, gs, ws = (i0, i1), (r0, r1), (g0, g1), (w0, w1)
        def idx_off(off):
            j = off // seg_out
            return j * seg_idx + base + (off - j * seg_out)

        for b in range(2):
            pltpu.sync_copy(
                idx_hbm.at[pl.ds(idx_off(wbase + b * chunk), chunk)], idxs[b])
            pltpu.async_copy(tbl_hbm.at[idxs[b]], rows[b], gs[b])

        def body(i, carry):
            for b in range(2):
                cur = 2 * i + b

                @pl.when(cur < n_it)
                def _():
                    off = wbase + cur * chunk
                    pltpu.make_async_copy(tbl_hbm.at[idxs[b]], rows[b],
                                          gs[b]).wait()
                    pltpu.async_copy(rows[b], out_hbm.at[pl.ds(off, chunk)],
                                     ws[b])

                    @pl.when(cur + 2 < n_it)
                    def _():
                        nxt = idx_off(wbase + (cur + 2) * chunk)
                        pltpu.sync_copy(idx_hbm.at[pl.ds(nxt, chunk)],
                                        idxs[b])
                        pltpu.make_async_copy(
                            rows[b], out_hbm.at[pl.ds(off, chunk)],
                            ws[b]).wait()
                        pltpu.async_copy(tbl_hbm.at[idxs[b]], rows[b], gs[b])

                    @pl.when(cur + 2 >= n_it)
                    def _():
                        pltpu.make_async_copy(
                            rows[b], out_hbm.at[pl.ds(off, chunk)],
                            ws[b]).wait()
            return carry

        lax.fori_loop(0, (n_it + 1) // 2, body, 0)

    return gather


def _att_body(*refs, nei, h, kd):
    msg_ref = refs[0]
    nrefs = refs[1:1 + nei]
    bg_ref = refs[1 + nei]
    wq_ref, wk_ref, wv_ref, wrep_ref = refs[2 + nei:6 + nei]
    out_ref = refs[-1]
    m = msg_ref[...]
    fb_in = jnp.where(m >= 0, m, 10.0 * m)
    q = jnp.dot(m.astype(jnp.bfloat16), wq_ref[...],
                preferred_element_type=jnp.float32)
    cols = lax.broadcasted_iota(jnp.int32, (h * kd, h), 0) // kd
    hh = lax.broadcasted_iota(jnp.int32, (h * kd, h), 1)
    s_mat = (cols == hh).astype(jnp.float32)
    s_t = (lax.broadcasted_iota(jnp.int32, (h, h * kd), 1) // kd
           == lax.broadcasted_iota(jnp.int32, (h, h * kd), 0)).astype(jnp.float32)
    bg = bg_ref[...]
    inv_sqrt = 1.0 / math.sqrt(kd)
    es, vs = [], []
    ssum = None
    for j in range(nei):
        nb = nrefs[j][...].astype(jnp.bfloat16)
        kj = jnp.dot(nb, wk_ref[...], preferred_element_type=jnp.float32)
        vj = jnp.dot(nb, wv_ref[...], preferred_element_type=jnp.float32)
        lj = jnp.dot(q * kj, s_mat, preferred_element_type=jnp.float32)
        mj = (bg[:, j:j + 1] != 0).astype(jnp.float32)
        ej = jnp.exp(lj * inv_sqrt) * mj
        ssum = ej if ssum is None else ssum + ej
        es.append(ej)
        vs.append(vj)
    inv = 1.0 / jnp.maximum(ssum, 1e-30)
    wsum = None
    for j in range(nei):
        aj = jnp.dot(es[j] * inv, s_t, preferred_element_type=jnp.float32)
        term = vs[j] * aj
        wsum = term if wsum is None else wsum + term
    nm = jnp.dot(wsum, wrep_ref[...], preferred_element_type=jnp.float32)
    out_ref[...] = _leaky(fb_in + nm)


def _out_body(fat_ref, ga_ref, wa_ref, wn_ref, bl_ref, out_ref):
    na = jnp.sum(ga_ref[...], axis=1)
    x = jnp.dot(fat_ref[...], wa_ref[...], preferred_element_type=jnp.float32)
    x = x + jnp.dot(na, wn_ref[...], preferred_element_type=jnp.float32)
    out_ref[...] = _leaky(x + bl_ref[...])


def kernel(fatoms, fbonds, agraph, bgraph, lig_scope,
           W_emb, b_emb, W_Q, W_K, W_V, W_out, W_last, b_last):
    del lig_scope
    e, fb_dim = fbonds.shape
    n, fa_dim = fatoms.shape
    nei = bgraph.shape[1]
    d = W_emb.shape[1]
    kd = W_out.shape[0]
    h = W_Q.shape[1] // kd
    bgraph = bgraph.astype(jnp.int32)
    agraph = agraph.astype(jnp.int32)

    b1 = 3200
    message, bgt = pl.pallas_call(
        _emb_body,
        grid=(e // b1,),
        in_specs=[pl.BlockSpec((b1, fb_dim), lambda i: (i, 0)),
                  pl.BlockSpec((fb_dim, d), lambda i: (0, 0)),
                  pl.BlockSpec((1, d), lambda i: (0, 0)),
                  pl.BlockSpec((b1, nei), lambda i: (i, 0))],
        out_specs=[pl.BlockSpec((b1, d), lambda i: (i, 0)),
                   pl.BlockSpec((nei, b1), lambda i: (0, i))],
        out_shape=[jax.ShapeDtypeStruct((e, d), jnp.float32),
                   jax.ShapeDtypeStruct((nei, e), jnp.int32)],
        compiler_params=pltpu.CompilerParams(dimension_semantics=("parallel",)),
    )(fbonds, W_emb, b_emb.reshape(1, d), bgraph)

    b2 = 1600
    wrep = jnp.tile(W_out, (h, 1)) / h
    att = functools.partial(_att_body, nei=nei, h=h, kd=kd)
    wqb = W_Q.astype(jnp.bfloat16)
    wkb = W_K.astype(jnp.bfloat16)
    wvb = W_V.astype(jnp.bfloat16)
    he = e // 2
    idx_full = bgt.reshape(-1)
    nei_halves = [
        _make_gather(he * nei, d, 200, jnp.float32,
                     seg_out=he, seg_idx=e, base=k * he)(message, idx_full)
        for k in range(2)]
    nblk = he // b2

    def att_half(k, nei_rows, alias):
        nei_specs = [pl.BlockSpec((b2, d),
                                  functools.partial(
                                      lambda j, i: (j * nblk + i, 0), j))
                     for j in range(nei)]
        extra_in, extra_arg, aliases = [], [], {}
        if alias is not None:
            extra_in = [pl.BlockSpec(memory_space=pl.ANY)]
            extra_arg = [alias]
            aliases = {6 + nei: 0}
        return pl.pallas_call(
            att,
            grid=(nblk,),
            in_specs=[pl.BlockSpec((b2, d), lambda i: (k * nblk + i, 0))]
                     + nei_specs
                     + [pl.BlockSpec((b2, nei), lambda i: (k * nblk + i, 0)),
                        pl.BlockSpec((d, h * kd), lambda i: (0, 0)),
                        pl.BlockSpec((d, h * kd), lambda i: (0, 0)),
                        pl.BlockSpec((d, h * kd), lambda i: (0, 0)),
                        pl.BlockSpec((h * kd, d), lambda i: (0, 0))]
                     + extra_in,
            out_specs=pl.BlockSpec((b2, d), lambda i: (k * nblk + i, 0)),
            out_shape=jax.ShapeDtypeStruct((e, d), jnp.float32),
            input_output_aliases=aliases,
            compiler_params=pltpu.CompilerParams(
                dimension_semantics=("parallel",)),
        )(message, *([nei_rows] * nei), bgraph, wqb, wkb, wvb, wrep,
          *extra_arg)

    m2_half = att_half(0, nei_halves[0], None)
    message2 = att_half(1, nei_halves[1], m2_half)

    n_pad = 10240
    ag_flat = jnp.concatenate(
        [agraph.reshape(-1),
         jnp.zeros((n_pad - n) * nei, jnp.int32)])
    gathered = _make_gather(n_pad * nei, d, 192, jnp.float32)(
        message2, ag_flat)

    b3 = 1000
    out = pl.pallas_call(
        _out_body,
        grid=(n // b3,),
        in_specs=[pl.BlockSpec((b3, fa_dim), lambda i: (i, 0)),
                  pl.BlockSpec((b3, nei, d), lambda i: (i, 0, 0)),
                  pl.BlockSpec((fa_dim, d), lambda i: (0, 0)),
                  pl.BlockSpec((d, d), lambda i: (0, 0)),
                  pl.BlockSpec((1, d), lambda i: (0, 0))],
        out_specs=pl.BlockSpec((b3, d), lambda i: (i, 0)),
        out_shape=jax.ShapeDtypeStruct((n, d), jnp.float32),
        compiler_params=pltpu.CompilerParams(dimension_semantics=("parallel",)),
    )(fatoms, gathered.reshape(n_pad, nei, d), W_last[:fa_dim],
      W_last[fa_dim:], b_last.reshape(1, d))
    return out


# speedup vs baseline: 1.1720x; 1.0510x over previous
import functools
import math

import jax
import jax.numpy as jnp
from jax import lax
from jax.experimental import pallas as pl
from jax.experimental.pallas import tpu as pltpu
from jax.experimental.pallas import tpu_sc as plsc

_NC, _NS = 2, 16
_NW = _NC * _NS


def _leaky(x):
    return jnp.where(x >= 0, x, 0.1 * x)


def _emb_body(fb_ref, we_ref, be_ref, bg_ref, msg_ref, bgt_ref):
    x = jnp.dot(fb_ref[...], we_ref[...], preferred_element_type=jnp.float32)
    msg_ref[...] = _leaky(x + be_ref[...])
    bgt_ref[...] = bg_ref[...].T


def _make_gather(total, d, chunk, dtype, seg_out=None, seg_idx=None, base=0):
    if seg_out is None:
        seg_out, seg_idx = total, total
    per_w = total // _NW
    n_it = per_w // chunk
    assert per_w * _NW == total and n_it * chunk == per_w and n_it >= 2
    assert seg_out % chunk == 0 and base % 8 == 0 and seg_idx % 8 == 0
    mesh = plsc.VectorSubcoreMesh(core_axis_name="c", subcore_axis_name="s",
                                  num_cores=_NC, num_subcores=_NS)

    @functools.partial(
        pl.kernel, mesh=mesh,
        out_type=jax.ShapeDtypeStruct((total, d), dtype),
        scratch_types=[pltpu.VMEM((chunk,), jnp.int32),
                       pltpu.VMEM((chunk,), jnp.int32),
                       pltpu.VMEM((chunk, d), dtype),
                       pltpu.VMEM((chunk, d), dtype),
                       pltpu.SemaphoreType.DMA,
                       pltpu.SemaphoreType.DMA,
                       pltpu.SemaphoreType.DMA,
                       pltpu.SemaphoreType.DMA],
    )
    def gather(tbl_hbm, idx_hbm, out_hbm, i0, i1, r0, r1, g0, g1, w0, w1):
        wid = lax.axis_index("s") * _NC + lax.axis_index("c")
        wbase = wid * per_w
        idxs, rows, gs, ws = (i0, i1), (r0, r1), (g0, g1), (w0, w1)
        def idx_off(off):
            j = off // seg_out
            return j * seg_idx + base + (off - j * seg_out)

        for b in range(2):
            pltpu.sync_copy(
                idx_hbm.at[pl.ds(idx_off(wbase + b * chunk), chunk)], idxs[b])
            pltpu.async_copy(tbl_hbm.at[idxs[b]], rows[b], gs[b])

        def body(i, carry):
            for b in range(2):
                cur = 2 * i + b

                @pl.when(cur < n_it)
                def _():
                    off = wbase + cur * chunk
                    pltpu.make_async_copy(tbl_hbm.at[idxs[b]], rows[b],
                                          gs[b]).wait()
                    pltpu.async_copy(rows[b], out_hbm.at[pl.ds(off, chunk)],
                                     ws[b])

                    @pl.when(cur + 2 < n_it)
                    def _():
                        nxt = idx_off(wbase + (cur + 2) * chunk)
                        pltpu.sync_copy(idx_hbm.at[pl.ds(nxt, chunk)],
                                        idxs[b])
                        pltpu.make_async_copy(
                            rows[b], out_hbm.at[pl.ds(off, chunk)],
                            ws[b]).wait()
                        pltpu.async_copy(tbl_hbm.at[idxs[b]], rows[b], gs[b])

                    @pl.when(cur + 2 >= n_it)
                    def _():
                        pltpu.make_async_copy(
                            rows[b], out_hbm.at[pl.ds(off, chunk)],
                            ws[b]).wait()
            return carry

        lax.fori_loop(0, (n_it + 1) // 2, body, 0)

    return gather


def _att_body(*refs, nei, h, kd):
    msg_ref = refs[0]
    nrefs = refs[1:1 + nei]
    bg_ref = refs[1 + nei]
    wq_ref, wk_ref, wv_ref, wrep_ref = refs[2 + nei:6 + nei]
    out_ref = refs[-1]
    m = msg_ref[...]
    fb_in = jnp.where(m >= 0, m, 10.0 * m)
    q = jnp.dot(m.astype(jnp.bfloat16), wq_ref[...],
                preferred_element_type=jnp.float32)
    cols = lax.broadcasted_iota(jnp.int32, (h * kd, h), 0) // kd
    hh = lax.broadcasted_iota(jnp.int32, (h * kd, h), 1)
    s_mat = (cols == hh).astype(jnp.float32)
    s_t = (lax.broadcasted_iota(jnp.int32, (h, h * kd), 1) // kd
           == lax.broadcasted_iota(jnp.int32, (h, h * kd), 0)).astype(jnp.float32)
    bg = bg_ref[...]
    inv_sqrt = 1.0 / math.sqrt(kd)
    es, vs = [], []
    ssum = None
    for j in range(nei):
        nb = nrefs[j][...].astype(jnp.bfloat16)
        kj = jnp.dot(nb, wk_ref[...], preferred_element_type=jnp.float32)
        vj = jnp.dot(nb, wv_ref[...], preferred_element_type=jnp.float32)
        lj = jnp.dot(q * kj, s_mat, preferred_element_type=jnp.float32)
        mj = (bg[:, j:j + 1] != 0).astype(jnp.float32)
        ej = jnp.exp(lj * inv_sqrt) * mj
        ssum = ej if ssum is None else ssum + ej
        es.append(ej)
        vs.append(vj)
    inv = 1.0 / jnp.maximum(ssum, 1e-30)
    wsum = None
    for j in range(nei):
        aj = jnp.dot(es[j] * inv, s_t, preferred_element_type=jnp.float32)
        term = vs[j] * aj
        wsum = term if wsum is None else wsum + term
    nm = jnp.dot(wsum, wrep_ref[...], preferred_element_type=jnp.float32)
    out_ref[...] = _leaky(fb_in + nm)


def _out_body(*refs, nei):
    fat_ref = refs[0]
    garefs = refs[1:1 + nei]
    wa_ref, wn_ref, bl_ref = refs[1 + nei:4 + nei]
    out_ref = refs[-1]
    na = garefs[0][...]
    for j in range(1, nei):
        na = na + garefs[j][...]
    x = jnp.dot(fat_ref[...], wa_ref[...], preferred_element_type=jnp.float32)
    x = x + jnp.dot(na, wn_ref[...], preferred_element_type=jnp.float32)
    out_ref[...] = _leaky(x + bl_ref[...])


def kernel(fatoms, fbonds, agraph, bgraph, lig_scope,
           W_emb, b_emb, W_Q, W_K, W_V, W_out, W_last, b_last):
    del lig_scope
    e, fb_dim = fbonds.shape
    n, fa_dim = fatoms.shape
    nei = bgraph.shape[1]
    d = W_emb.shape[1]
    kd = W_out.shape[0]
    h = W_Q.shape[1] // kd
    bgraph = bgraph.astype(jnp.int32)
    agraph = agraph.astype(jnp.int32)

    b1 = 3200
    message, bgt = pl.pallas_call(
        _emb_body,
        grid=(e // b1,),
        in_specs=[pl.BlockSpec((b1, fb_dim), lambda i: (i, 0)),
                  pl.BlockSpec((fb_dim, d), lambda i: (0, 0)),
                  pl.BlockSpec((1, d), lambda i: (0, 0)),
                  pl.BlockSpec((b1, nei), lambda i: (i, 0))],
        out_specs=[pl.BlockSpec((b1, d), lambda i: (i, 0)),
                   pl.BlockSpec((nei, b1), lambda i: (0, i))],
        out_shape=[jax.ShapeDtypeStruct((e, d), jnp.float32),
                   jax.ShapeDtypeStruct((nei, e), jnp.int32)],
        compiler_params=pltpu.CompilerParams(dimension_semantics=("parallel",)),
    )(fbonds, W_emb, b_emb.reshape(1, d), bgraph)

    b2 = 1600
    wrep = jnp.tile(W_out, (h, 1)) / h
    att = functools.partial(_att_body, nei=nei, h=h, kd=kd)
    wqb = W_Q.astype(jnp.bfloat16)
    wkb = W_K.astype(jnp.bfloat16)
    wvb = W_V.astype(jnp.bfloat16)
    he = e // 2
    idx_full = bgt.reshape(-1)
    nei_halves = [
        _make_gather(he * nei, d, 200, jnp.float32,
                     seg_out=he, seg_idx=e, base=k * he)(message, idx_full)
        for k in range(2)]
    nblk = he // b2

    def att_half(k, nei_rows, alias):
        nei_specs = [pl.BlockSpec((b2, d),
                                  functools.partial(
                                      lambda j, i: (j * nblk + i, 0), j))
                     for j in range(nei)]
        extra_in, extra_arg, aliases = [], [], {}
        if alias is not None:
            extra_in = [pl.BlockSpec(memory_space=pl.ANY)]
            extra_arg = [alias]
            aliases = {6 + nei: 0}
        return pl.pallas_call(
            att,
            grid=(nblk,),
            in_specs=[pl.BlockSpec((b2, d), lambda i: (k * nblk + i, 0))]
                     + nei_specs
                     + [pl.BlockSpec((b2, nei), lambda i: (k * nblk + i, 0)),
                        pl.BlockSpec((d, h * kd), lambda i: (0, 0)),
                        pl.BlockSpec((d, h * kd), lambda i: (0, 0)),
                        pl.BlockSpec((d, h * kd), lambda i: (0, 0)),
                        pl.BlockSpec((h * kd, d), lambda i: (0, 0))]
                     + extra_in,
            out_specs=pl.BlockSpec((b2, d), lambda i: (k * nblk + i, 0)),
            out_shape=jax.ShapeDtypeStruct((e, d), jnp.float32),
            input_output_aliases=aliases,
            compiler_params=pltpu.CompilerParams(
                dimension_semantics=("parallel",)),
        )(message, *([nei_rows] * nei), bgraph, wqb, wkb, wvb, wrep,
          *extra_arg)

    m2_half = att_half(0, nei_halves[0], None)
    message2 = att_half(1, nei_halves[1], m2_half)

    n_pad = 10240
    agt = jnp.zeros((nei, n_pad), jnp.int32).at[:, :n].set(agraph.T)
    gathered = _make_gather(n_pad * nei, d, 192, jnp.float32)(
        message2, agt.reshape(-1))

    b3 = 1024
    npb = n_pad // b3
    fat_pad = jnp.concatenate(
        [fatoms, jnp.zeros((n_pad - n, fa_dim), fatoms.dtype)])
    ga_specs = [pl.BlockSpec((b3, d),
                             functools.partial(lambda j, i: (j * npb + i, 0), j))
                for j in range(nei)]
    out = pl.pallas_call(
        functools.partial(_out_body, nei=nei),
        grid=(npb,),
        in_specs=[pl.BlockSpec((b3, fa_dim), lambda i: (i, 0))]
                 + ga_specs
                 + [pl.BlockSpec((fa_dim, d), lambda i: (0, 0)),
                    pl.BlockSpec((d, d), lambda i: (0, 0)),
                    pl.BlockSpec((1, d), lambda i: (0, 0))],
        out_specs=pl.BlockSpec((b3, d), lambda i: (i, 0)),
        out_shape=jax.ShapeDtypeStruct((n_pad, d), jnp.float32),
        compiler_params=pltpu.CompilerParams(dimension_semantics=("parallel",)),
    )(fat_pad, *([gathered] * nei), W_last[:fa_dim],
      W_last[fa_dim:], b_last.reshape(1, d))
    return out[:n]
